# baseline (device time: 28534 ns/iter reference)
import jax
import jax.numpy as jnp
from jax import lax
from jax.experimental import pallas as pl
from jax.experimental.pallas import tpu as pltpu

N_DEV = 4
ST = 64


def kernel(A, B):
    m, _ = A.shape
    _, n = B.shape
    halfm = m // 2
    q = halfm // 2
    e = q // 2

    def body(a_ref, b_ref, out_ref, a_bf, b_bf,
             stg1_A, rbuf1_A, stg2_A, rbuf2_A, stg3_A, rbuf3_A, rbuf4_A,
             stg1_B, rbuf1_B, stg2_B, rbuf2_B, stg3_B, rbuf3_B, rbuf4_B,
             send_x, recv_x, send_y, recv_y):
        my = lax.axis_index("i")
        mx = lax.div(my, 2)
        yy = jnp.bitwise_xor(lax.rem(my, 2), mx)
        px = 3 - my
        py = jnp.bitwise_xor(my, 1)

        barrier_sem = pltpu.get_barrier_semaphore()
        for nbr in (px, py):
            pl.semaphore_signal(
                barrier_sem, inc=1,
                device_id=(nbr,), device_id_type=pl.DeviceIdType.MESH,
            )
        pl.semaphore_wait(barrier_sem, 2)

        a_bf[:, :] = a_ref[:, :].astype(jnp.bfloat16)
        b_bf[:, :] = b_ref[:, :].astype(jnp.bfloat16)

        def dot_rows(r0, nrows):
            return jnp.dot(
                a_bf[pl.ds(r0, nrows), :], b_bf[:, :],
                preferred_element_type=jnp.float32,
            )

        def xch(link, slot, src, dst, partner):
            send, recv = (send_x, recv_x) if link == "x" else (send_y, recv_y)
            return pltpu.make_async_remote_copy(
                src_ref=src, dst_ref=dst,
                send_sem=send.at[slot], recv_sem=recv.at[slot],
                device_id=(partner,), device_id_type=pl.DeviceIdType.MESH,
            )

        def uoff_A(k):
            return lax.rem(2 * (1 - yy) + k, N_DEV) * ST

        def uoff_B(k):
            return lax.rem(2 * (1 - mx) + k, N_DEV) * ST

        qA_keep = mx * q
        qA_sent = (1 - mx) * q
        qB_keep = halfm + yy * q
        qB_sent = halfm + (1 - yy) * q

        for k in range(4):
            uA = uoff_A(k)
            stg1_A[pl.ds(uA, ST), :] = dot_rows(qA_sent + uA, ST).astype(
                jnp.bfloat16
            )
            xch("x", k, stg1_A.at[pl.ds(uA, ST), :],
                rbuf1_A.at[pl.ds(uA, ST), :], px).start()
            uB = uoff_B(k)
            stg1_B[pl.ds(uB, ST), :] = dot_rows(qB_sent + uB, ST).astype(
                jnp.bfloat16
            )
            xch("y", k, stg1_B.at[pl.ds(uB, ST), :],
                rbuf1_B.at[pl.ds(uB, ST), :], py).start()

        out_ref[pl.ds(qA_keep, q), :] = dot_rows(qA_keep, q)
        out_ref[pl.ds(qB_keep, q), :] = dot_rows(qB_keep, q)

        for k in range(2):
            uA = uoff_A(k)
            xch("x", k, stg1_A.at[pl.ds(uA, ST), :],
                rbuf1_A.at[pl.ds(uA, ST), :], px).wait_recv()
            stg2_A[pl.ds(k * ST, ST), :] = (
                out_ref[pl.ds(qA_keep + uA, ST), :].astype(jnp.bfloat16)
                + rbuf1_A[pl.ds(uA, ST), :]
            )
            xch("y", 4 + k, stg2_A.at[pl.ds(k * ST, ST), :],
                rbuf2_A.at[pl.ds(k * ST, ST), :], py).start()
            uB = uoff_B(k)
            xch("y", k, stg1_B.at[pl.ds(uB, ST), :],
                rbuf1_B.at[pl.ds(uB, ST), :], py).wait_recv()
            stg2_B[pl.ds(k * ST, ST), :] = (
                out_ref[pl.ds(qB_keep + uB, ST), :].astype(jnp.bfloat16)
                + rbuf1_B[pl.ds(uB, ST), :]
            )
            xch("x", 4 + k, stg2_B.at[pl.ds(k * ST, ST), :],
                rbuf2_B.at[pl.ds(k * ST, ST), :], px).start()
        for k in range(2, 4):
            uA = uoff_A(k)
            xch("x", k, stg1_A.at[pl.ds(uA, ST), :],
                rbuf1_A.at[pl.ds(uA, ST), :], px).wait_recv()
            uB = uoff_B(k)
            xch("y", k, stg1_B.at[pl.ds(uB, ST), :],
                rbuf1_B.at[pl.ds(uB, ST), :], py).wait_recv()

        for v in range(2):
            uA = uoff_A(2 + v)
            rowsA = pl.ds(qA_keep + uA, ST)
            xch("y", 4 + v, stg2_A.at[pl.ds(v * ST, ST), :],
                rbuf2_A.at[pl.ds(v * ST, ST), :], py).wait_recv()
            redA = jnp.maximum(
                out_ref[rowsA, :]
                + rbuf1_A[pl.ds(uA, ST), :].astype(jnp.float32)
                + rbuf2_A[pl.ds(v * ST, ST), :].astype(jnp.float32),
                0.0,
            )
            out_ref[rowsA, :] = redA
            stg3_A[pl.ds(v * ST, ST), :] = redA.astype(jnp.bfloat16)
            xch("y", 6 + v, stg3_A.at[pl.ds(v * ST, ST), :],
                rbuf3_A.at[pl.ds(v * ST, ST), :], py).start()
            xch("x", 8 + v, stg3_A.at[pl.ds(v * ST, ST), :],
                rbuf4_A.at[pl.ds(uA, ST), :], px).start()

            uB = uoff_B(2 + v)
            rowsB = pl.ds(qB_keep + uB, ST)
            xch("x", 4 + v, stg2_B.at[pl.ds(v * ST, ST), :],
                rbuf2_B.at[pl.ds(v * ST, ST), :], px).wait_recv()
            redB = jnp.maximum(
                out_ref[rowsB, :]
                + rbuf1_B[pl.ds(uB, ST), :].astype(jnp.float32)
                + rbuf2_B[pl.ds(v * ST, ST), :].astype(jnp.float32),
                0.0,
            )
            out_ref[rowsB, :] = redB
            stg3_B[pl.ds(v * ST, ST), :] = redB.astype(jnp.bfloat16)
            xch("x", 6 + v, stg3_B.at[pl.ds(v * ST, ST), :],
                rbuf3_B.at[pl.ds(v * ST, ST), :], px).start()
            xch("y", 8 + v, stg3_B.at[pl.ds(v * ST, ST), :],
                rbuf4_B.at[pl.ds(uB, ST), :], py).start()

        for v in range(2):
            uA = uoff_A(v)
            xch("y", 6 + v, stg3_A.at[pl.ds(v * ST, ST), :],
                rbuf3_A.at[pl.ds(v * ST, ST), :], py).wait_recv()
            xch("x", 10 + v, rbuf3_A.at[pl.ds(v * ST, ST), :],
                rbuf4_A.at[pl.ds(uA, ST), :], px).start()
            out_ref[pl.ds(qA_keep + uA, ST), :] = (
                rbuf3_A[pl.ds(v * ST, ST), :].astype(jnp.float32)
            )
            uB = uoff_B(v)
            xch("x", 6 + v, stg3_B.at[pl.ds(v * ST, ST), :],
                rbuf3_B.at[pl.ds(v * ST, ST), :], px).wait_recv()
            xch("y", 10 + v, rbuf3_B.at[pl.ds(v * ST, ST), :],
                rbuf4_B.at[pl.ds(uB, ST), :], py).start()
            out_ref[pl.ds(qB_keep + uB, ST), :] = (
                rbuf3_B[pl.ds(v * ST, ST), :].astype(jnp.float32)
            )

        for k in range(4):
            uA = uoff_A(2 + k) if k < 2 else uoff_A(k - 2)
            xch("x", 8 + k, stg3_A.at[pl.ds(0, ST), :],
                rbuf4_A.at[pl.ds(uA, ST), :], px).wait_recv()
            out_ref[pl.ds(qA_sent + uA, ST), :] = (
                rbuf4_A[pl.ds(uA, ST), :].astype(jnp.float32)
            )
            uB = uoff_B(2 + k) if k < 2 else uoff_B(k - 2)
            xch("y", 8 + k, stg3_B.at[pl.ds(0, ST), :],
                rbuf4_B.at[pl.ds(uB, ST), :], py).wait_recv()
            out_ref[pl.ds(qB_sent + uB, ST), :] = (
                rbuf4_B[pl.ds(uB, ST), :].astype(jnp.float32)
            )

        for slot in range(12):
            xch("x", slot, stg1_A.at[pl.ds(0, ST), :],
                rbuf1_A.at[pl.ds(0, ST), :], px).wait_send()
            xch("y", slot, stg1_B.at[pl.ds(0, ST), :],
                rbuf1_B.at[pl.ds(0, ST), :], py).wait_send()

    return pl.pallas_call(
        body,
        out_shape=jax.ShapeDtypeStruct((m, n), jnp.float32),
        in_specs=[
            pl.BlockSpec(memory_space=pltpu.VMEM),
            pl.BlockSpec(memory_space=pltpu.VMEM),
        ],
        out_specs=pl.BlockSpec(memory_space=pltpu.VMEM),
        scratch_shapes=[
            pltpu.VMEM(A.shape, jnp.bfloat16),
            pltpu.VMEM(B.shape, jnp.bfloat16),
            pltpu.VMEM((q, n), jnp.bfloat16),
            pltpu.VMEM((q, n), jnp.bfloat16),
            pltpu.VMEM((e, n), jnp.bfloat16),
            pltpu.VMEM((e, n), jnp.bfloat16),
            pltpu.VMEM((e, n), jnp.bfloat16),
            pltpu.VMEM((e, n), jnp.bfloat16),
            pltpu.VMEM((q, n), jnp.bfloat16),
            pltpu.VMEM((q, n), jnp.bfloat16),
            pltpu.VMEM((q, n), jnp.bfloat16),
            pltpu.VMEM((e, n), jnp.bfloat16),
            pltpu.VMEM((e, n), jnp.bfloat16),
            pltpu.VMEM((e, n), jnp.bfloat16),
            pltpu.VMEM((e, n), jnp.bfloat16),
            pltpu.VMEM((q, n), jnp.bfloat16),
            pltpu.SemaphoreType.DMA((12,)),
            pltpu.SemaphoreType.DMA((12,)),
            pltpu.SemaphoreType.DMA((12,)),
            pltpu.SemaphoreType.DMA((12,)),
        ],
        compiler_params=pltpu.CompilerParams(collective_id=0),
    )(A, B)


# device time: 28449 ns/iter; 1.0030x vs baseline; 1.0030x over previous
import jax
import jax.numpy as jnp
from jax import lax
from jax.experimental import pallas as pl
from jax.experimental.pallas import tpu as pltpu

N_DEV = 4
ST = 64


def kernel(A, B):
    m, _ = A.shape
    _, n = B.shape
    halfm = m // 2
    q = halfm // 2
    e = q // 2

    def body(a_ref, b_ref, out_ref, b_bf,
             stg1_A, rbuf1_A, stg2_A, rbuf2_A, stg3_A, rbuf3_A, rbuf4_A,
             stg1_B, rbuf1_B, stg2_B, rbuf2_B, stg3_B, rbuf3_B, rbuf4_B,
             send_x, recv_x, send_y, recv_y):
        my = lax.axis_index("i")
        mx = lax.div(my, 2)
        yy = jnp.bitwise_xor(lax.rem(my, 2), mx)
        px = 3 - my
        py = jnp.bitwise_xor(my, 1)

        barrier_sem = pltpu.get_barrier_semaphore()
        for nbr in (px, py):
            pl.semaphore_signal(
                barrier_sem, inc=1,
                device_id=(nbr,), device_id_type=pl.DeviceIdType.MESH,
            )
        pl.semaphore_wait(barrier_sem, 2)

        b_bf[:, :] = b_ref[:, :].astype(jnp.bfloat16)

        def dot_rows(r0, nrows):
            return jnp.dot(
                a_ref[pl.ds(r0, nrows), :].astype(jnp.bfloat16), b_bf[:, :],
                preferred_element_type=jnp.float32,
            )

        def xch(link, slot, src, dst, partner):
            send, recv = (send_x, recv_x) if link == "x" else (send_y, recv_y)
            return pltpu.make_async_remote_copy(
                src_ref=src, dst_ref=dst,
                send_sem=send.at[slot], recv_sem=recv.at[slot],
                device_id=(partner,), device_id_type=pl.DeviceIdType.MESH,
            )

        def uoff_A(k):
            return lax.rem(2 * (1 - yy) + k, N_DEV) * ST

        def uoff_B(k):
            return lax.rem(2 * (1 - mx) + k, N_DEV) * ST

        qA_keep = mx * q
        qA_sent = (1 - mx) * q
        qB_keep = halfm + yy * q
        qB_sent = halfm + (1 - yy) * q

        for k in range(4):
            uA = uoff_A(k)
            stg1_A[pl.ds(uA, ST), :] = dot_rows(qA_sent + uA, ST).astype(
                jnp.bfloat16
            )
            xch("x", k, stg1_A.at[pl.ds(uA, ST), :],
                rbuf1_A.at[pl.ds(uA, ST), :], px).start()
            uB = uoff_B(k)
            stg1_B[pl.ds(uB, ST), :] = dot_rows(qB_sent + uB, ST).astype(
                jnp.bfloat16
            )
            xch("y", k, stg1_B.at[pl.ds(uB, ST), :],
                rbuf1_B.at[pl.ds(uB, ST), :], py).start()

        out_ref[pl.ds(qA_keep, q), :] = dot_rows(qA_keep, q)
        out_ref[pl.ds(qB_keep, q), :] = dot_rows(qB_keep, q)

        for k in range(2):
            uA = uoff_A(k)
            xch("x", k, stg1_A.at[pl.ds(uA, ST), :],
                rbuf1_A.at[pl.ds(uA, ST), :], px).wait_recv()
            stg2_A[pl.ds(k * ST, ST), :] = (
                out_ref[pl.ds(qA_keep + uA, ST), :].astype(jnp.bfloat16)
                + rbuf1_A[pl.ds(uA, ST), :]
            )
            xch("y", 4 + k, stg2_A.at[pl.ds(k * ST, ST), :],
                rbuf2_A.at[pl.ds(k * ST, ST), :], py).start()
            uB = uoff_B(k)
            xch("y", k, stg1_B.at[pl.ds(uB, ST), :],
                rbuf1_B.at[pl.ds(uB, ST), :], py).wait_recv()
            stg2_B[pl.ds(k * ST, ST), :] = (
                out_ref[pl.ds(qB_keep + uB, ST), :].astype(jnp.bfloat16)
                + rbuf1_B[pl.ds(uB, ST), :]
            )
            xch("x", 4 + k, stg2_B.at[pl.ds(k * ST, ST), :],
                rbuf2_B.at[pl.ds(k * ST, ST), :], px).start()
        for k in range(2, 4):
            uA = uoff_A(k)
            xch("x", k, stg1_A.at[pl.ds(uA, ST), :],
                rbuf1_A.at[pl.ds(uA, ST), :], px).wait_recv()
            uB = uoff_B(k)
            xch("y", k, stg1_B.at[pl.ds(uB, ST), :],
                rbuf1_B.at[pl.ds(uB, ST), :], py).wait_recv()

        for v in range(2):
            uA = uoff_A(2 + v)
            rowsA = pl.ds(qA_keep + uA, ST)
            xch("y", 4 + v, stg2_A.at[pl.ds(v * ST, ST), :],
                rbuf2_A.at[pl.ds(v * ST, ST), :], py).wait_recv()
            redA = jnp.maximum(
                out_ref[rowsA, :]
                + rbuf1_A[pl.ds(uA, ST), :].astype(jnp.float32)
                + rbuf2_A[pl.ds(v * ST, ST), :].astype(jnp.float32),
                0.0,
            )
            out_ref[rowsA, :] = redA
            stg3_A[pl.ds(v * ST, ST), :] = redA.astype(jnp.bfloat16)
            xch("y", 6 + v, stg3_A.at[pl.ds(v * ST, ST), :],
                rbuf3_A.at[pl.ds(v * ST, ST), :], py).start()
            xch("x", 8 + v, stg3_A.at[pl.ds(v * ST, ST), :],
                rbuf4_A.at[pl.ds(uA, ST), :], px).start()

            uB = uoff_B(2 + v)
            rowsB = pl.ds(qB_keep + uB, ST)
            xch("x", 4 + v, stg2_B.at[pl.ds(v * ST, ST), :],
                rbuf2_B.at[pl.ds(v * ST, ST), :], px).wait_recv()
            redB = jnp.maximum(
                out_ref[rowsB, :]
                + rbuf1_B[pl.ds(uB, ST), :].astype(jnp.float32)
                + rbuf2_B[pl.ds(v * ST, ST), :].astype(jnp.float32),
                0.0,
            )
            out_ref[rowsB, :] = redB
            stg3_B[pl.ds(v * ST, ST), :] = redB.astype(jnp.bfloat16)
            xch("x", 6 + v, stg3_B.at[pl.ds(v * ST, ST), :],
                rbuf3_B.at[pl.ds(v * ST, ST), :], px).start()
            xch("y", 8 + v, stg3_B.at[pl.ds(v * ST, ST), :],
                rbuf4_B.at[pl.ds(uB, ST), :], py).start()

        for v in range(2):
            uA = uoff_A(v)
            xch("y", 6 + v, stg3_A.at[pl.ds(v * ST, ST), :],
                rbuf3_A.at[pl.ds(v * ST, ST), :], py).wait_recv()
            xch("x", 10 + v, rbuf3_A.at[pl.ds(v * ST, ST), :],
                rbuf4_A.at[pl.ds(uA, ST), :], px).start()
            out_ref[pl.ds(qA_keep + uA, ST), :] = (
                rbuf3_A[pl.ds(v * ST, ST), :].astype(jnp.float32)
            )
            uB = uoff_B(v)
            xch("x", 6 + v, stg3_B.at[pl.ds(v * ST, ST), :],
                rbuf3_B.at[pl.ds(v * ST, ST), :], px).wait_recv()
            xch("y", 10 + v, rbuf3_B.at[pl.ds(v * ST, ST), :],
                rbuf4_B.at[pl.ds(uB, ST), :], py).start()
            out_ref[pl.ds(qB_keep + uB, ST), :] = (
                rbuf3_B[pl.ds(v * ST, ST), :].astype(jnp.float32)
            )

        for k in range(4):
            uA = uoff_A(2 + k) if k < 2 else uoff_A(k - 2)
            xch("x", 8 + k, stg3_A.at[pl.ds(0, ST), :],
                rbuf4_A.at[pl.ds(uA, ST), :], px).wait_recv()
            out_ref[pl.ds(qA_sent + uA, ST), :] = (
                rbuf4_A[pl.ds(uA, ST), :].astype(jnp.float32)
            )
            uB = uoff_B(2 + k) if k < 2 else uoff_B(k - 2)
            xch("y", 8 + k, stg3_B.at[pl.ds(0, ST), :],
                rbuf4_B.at[pl.ds(uB, ST), :], py).wait_recv()
            out_ref[pl.ds(qB_sent + uB, ST), :] = (
                rbuf4_B[pl.ds(uB, ST), :].astype(jnp.float32)
            )

        for slot in range(12):
            xch("x", slot, stg1_A.at[pl.ds(0, ST), :],
                rbuf1_A.at[pl.ds(0, ST), :], px).wait_send()
            xch("y", slot, stg1_B.at[pl.ds(0, ST), :],
                rbuf1_B.at[pl.ds(0, ST), :], py).wait_send()

    return pl.pallas_call(
        body,
        out_shape=jax.ShapeDtypeStruct((m, n), jnp.float32),
        in_specs=[
            pl.BlockSpec(memory_space=pltpu.VMEM),
            pl.BlockSpec(memory_space=pltpu.VMEM),
        ],
        out_specs=pl.BlockSpec(memory_space=pltpu.VMEM),
        scratch_shapes=[
            pltpu.VMEM(B.shape, jnp.bfloat16),
            pltpu.VMEM((q, n), jnp.bfloat16),
            pltpu.VMEM((q, n), jnp.bfloat16),
            pltpu.VMEM((e, n), jnp.bfloat16),
            pltpu.VMEM((e, n), jnp.bfloat16),
            pltpu.VMEM((e, n), jnp.bfloat16),
            pltpu.VMEM((e, n), jnp.bfloat16),
            pltpu.VMEM((q, n), jnp.bfloat16),
            pltpu.VMEM((q, n), jnp.bfloat16),
            pltpu.VMEM((q, n), jnp.bfloat16),
            pltpu.VMEM((e, n), jnp.bfloat16),
            pltpu.VMEM((e, n), jnp.bfloat16),
            pltpu.VMEM((e, n), jnp.bfloat16),
            pltpu.VMEM((e, n), jnp.bfloat16),
            pltpu.VMEM((q, n), jnp.bfloat16),
            pltpu.SemaphoreType.DMA((12,)),
            pltpu.SemaphoreType.DMA((12,)),
            pltpu.SemaphoreType.DMA((12,)),
            pltpu.SemaphoreType.DMA((12,)),
        ],
        compiler_params=pltpu.CompilerParams(collective_id=0),
    )(A, B)


# device time: 28312 ns/iter; 1.0078x vs baseline; 1.0048x over previous
import jax
import jax.numpy as jnp
from jax import lax
from jax.experimental import pallas as pl
from jax.experimental.pallas import tpu as pltpu

N_DEV = 4
SUB = 4


def kernel(A, B):
    m, _ = A.shape
    _, n = B.shape
    chunk = m // N_DEV
    half = chunk // 2
    subh = half // SUB

    def body(a_ref, b_ref, out_ref,
             a_bf, b_bf,
             stage_r, comm_r, stage_l, comm_l,
             ag_stage_r, ag_comm_r, ag_stage_l, ag_comm_l,
             send_r, recv_r, send_l, recv_l):
        my = lax.axis_index("i")
        left = lax.rem(my + N_DEV - 1, N_DEV)
        right = lax.rem(my + 1, N_DEV)

        barrier_sem = pltpu.get_barrier_semaphore()
        for nbr in (left, right):
            pl.semaphore_signal(
                barrier_sem, inc=1,
                device_id=(nbr,), device_id_type=pl.DeviceIdType.MESH,
            )
        pl.semaphore_wait(barrier_sem, 2)

        a_bf[:, :] = a_ref[:, :].astype(jnp.bfloat16)
        b_bf[:, :] = b_ref[:, :].astype(jnp.bfloat16)

        def compute_block(c):
            rows = pl.ds(c * chunk, chunk)
            out_ref[rows, :] = jnp.dot(
                a_bf[rows, :], b_bf[:, :],
                preferred_element_type=jnp.float32,
            )

        sub = lambda u: slice(u * subh, (u + 1) * subh)

        def rs_rdma(d, s, u):
            slot = s * SUB + u
            if d == "r":
                return pltpu.make_async_remote_copy(
                    src_ref=stage_r.at[s, sub(u), :],
                    dst_ref=comm_r.at[s, sub(u), :],
                    send_sem=send_r.at[slot], recv_sem=recv_r.at[slot],
                    device_id=(right,), device_id_type=pl.DeviceIdType.MESH,
                )
            return pltpu.make_async_remote_copy(
                src_ref=stage_l.at[s, sub(u), :],
                dst_ref=comm_l.at[s, sub(u), :],
                send_sem=send_l.at[slot], recv_sem=recv_l.at[slot],
                device_id=(left,), device_id_type=pl.DeviceIdType.MESH,
            )

        def ag_rdma(d, s, u):
            slot = (N_DEV - 1) * SUB + s * SUB + u
            if d == "r":
                src = ag_stage_r if s == 0 else ag_comm_r.at[s - 1]
                return pltpu.make_async_remote_copy(
                    src_ref=src.at[sub(u), :],
                    dst_ref=ag_comm_r.at[s, sub(u), :],
                    send_sem=send_r.at[slot], recv_sem=recv_r.at[slot],
                    device_id=(right,), device_id_type=pl.DeviceIdType.MESH,
                )
            src = ag_stage_l if s == 0 else ag_comm_l.at[s - 1]
            return pltpu.make_async_remote_copy(
                src_ref=src.at[sub(u), :],
                dst_ref=ag_comm_l.at[s, sub(u), :],
                send_sem=send_l.at[slot], recv_sem=recv_l.at[slot],
                device_id=(left,), device_id_type=pl.DeviceIdType.MESH,
            )

        top0 = pl.ds(my * chunk, half)
        bot0 = pl.ds(my * chunk + half, half)
        stage_r[0] = jnp.dot(
            a_bf[top0, :], b_bf[:, :], preferred_element_type=jnp.float32
        ).astype(jnp.bfloat16)
        stage_l[0] = jnp.dot(
            a_bf[bot0, :], b_bf[:, :], preferred_element_type=jnp.float32
        ).astype(jnp.bfloat16)
        for u in range(SUB):
            rs_rdma("r", 0, u).start()
            rs_rdma("l", 0, u).start()

        compute_block(lax.rem(my + N_DEV - 1, N_DEV))
        compute_block(lax.rem(my + 1, N_DEV))

        for s in range(N_DEV - 1):
            rc_r = lax.rem(my - s - 1 + N_DEV, N_DEV)
            rc_l = lax.rem(my + s + 1, N_DEV)
            for u in range(SUB):
                rows_rt = pl.ds(rc_r * chunk + u * subh, subh)
                rows_lb = pl.ds(rc_l * chunk + half + u * subh, subh)
                rs_rdma("r", s, u).wait()
                if s < N_DEV - 2:
                    stage_r[s + 1, sub(u), :] = (
                        comm_r[s, sub(u), :]
                        + out_ref[rows_rt, :].astype(jnp.bfloat16)
                    )
                    rs_rdma("r", s + 1, u).start()
                else:
                    red = jnp.maximum(
                        out_ref[rows_rt, :]
                        + comm_r[s, sub(u), :].astype(jnp.float32),
                        0.0,
                    )
                    out_ref[rows_rt, :] = red
                    ag_stage_r[sub(u), :] = red.astype(jnp.bfloat16)
                    ag_rdma("r", 0, u).start()
                rs_rdma("l", s, u).wait()
                if s < N_DEV - 2:
                    stage_l[s + 1, sub(u), :] = (
                        comm_l[s, sub(u), :]
                        + out_ref[rows_lb, :].astype(jnp.bfloat16)
                    )
                    rs_rdma("l", s + 1, u).start()
                else:
                    red = jnp.maximum(
                        out_ref[rows_lb, :]
                        + comm_l[s, sub(u), :].astype(jnp.float32),
                        0.0,
                    )
                    out_ref[rows_lb, :] = red
                    ag_stage_l[sub(u), :] = red.astype(jnp.bfloat16)
                    ag_rdma("l", 0, u).start()
            if s == 0:
                compute_block(lax.rem(my + 2, N_DEV))

        for s in range(N_DEV - 1):
            rc_r = lax.rem(my - s + N_DEV, N_DEV)
            rc_l = lax.rem(my + s, N_DEV)
            for u in range(SUB):
                ag_rdma("r", s, u).wait()
                if s < N_DEV - 2:
                    ag_rdma("r", s + 1, u).start()
                out_ref[pl.ds(rc_r * chunk + u * subh, subh), :] = (
                    ag_comm_r[s, sub(u), :].astype(jnp.float32)
                )
                ag_rdma("l", s, u).wait()
                if s < N_DEV - 2:
                    ag_rdma("l", s + 1, u).start()
                out_ref[pl.ds(rc_l * chunk + half + u * subh, subh), :] = (
                    ag_comm_l[s, sub(u), :].astype(jnp.float32)
                )

    n_slots = 2 * (N_DEV - 1) * SUB
    return pl.pallas_call(
        body,
        out_shape=jax.ShapeDtypeStruct((m, n), jnp.float32),
        in_specs=[
            pl.BlockSpec(memory_space=pltpu.VMEM),
            pl.BlockSpec(memory_space=pltpu.VMEM),
        ],
        out_specs=pl.BlockSpec(memory_space=pltpu.VMEM),
        scratch_shapes=[
            pltpu.VMEM(A.shape, jnp.bfloat16),
            pltpu.VMEM(B.shape, jnp.bfloat16),
            pltpu.VMEM((N_DEV - 1, half, n), jnp.bfloat16),
            pltpu.VMEM((N_DEV - 1, half, n), jnp.bfloat16),
            pltpu.VMEM((N_DEV - 1, half, n), jnp.bfloat16),
            pltpu.VMEM((N_DEV - 1, half, n), jnp.bfloat16),
            pltpu.VMEM((half, n), jnp.bfloat16),
            pltpu.VMEM((N_DEV - 1, half, n), jnp.bfloat16),
            pltpu.VMEM((half, n), jnp.bfloat16),
            pltpu.VMEM((N_DEV - 1, half, n), jnp.bfloat16),
            pltpu.SemaphoreType.DMA((n_slots,)),
            pltpu.SemaphoreType.DMA((n_slots,)),
            pltpu.SemaphoreType.DMA((n_slots,)),
            pltpu.SemaphoreType.DMA((n_slots,)),
        ],
        compiler_params=pltpu.CompilerParams(collective_id=0),
    )(A, B)
